# baseline (speedup 1.0000x reference)
import functools

import jax
import jax.numpy as jnp
from jax import lax
from jax.experimental import pallas as pl
from jax.experimental.pallas import tpu as pltpu
from jax.experimental.pallas import tpu_sc as plsc

B = 4
CAP = 256
IMG_LEN = 128 * 128
SEQ = CAP + 2 * IMG_LEN
N = B * SEQ
D0, D1, D2 = 32, 48, 48
DT = D0 + D1 + D2
V0, V1, V2 = 1024, 512, 512

NW = 32
PER_W = N // NW
M = 344
MP = 352
G = MP // 16
STEPS = PER_W // M


@functools.partial(
    pl.kernel,
    out_type=jax.ShapeDtypeStruct((N, DT), jnp.float32),
    mesh=plsc.VectorSubcoreMesh(core_axis_name="c", subcore_axis_name="s"),
    compiler_params=pltpu.CompilerParams(use_tc_tiling_on_sc=False,
                                         needs_layout_passes=False),
    scratch_types=[
        pltpu.VMEM_SHARED((V0, D0), jnp.float32),
        pltpu.VMEM_SHARED((V1, D1), jnp.float32),
        pltpu.VMEM_SHARED((V2, D2), jnp.float32),
        [pltpu.VMEM((3 * MP,), jnp.int32) for _ in range(2)],
        [pltpu.VMEM((MP,), jnp.int32) for _ in range(6)],
        [pltpu.VMEM((M, D0), jnp.float32) for _ in range(2)],
        [pltpu.VMEM((M, D1), jnp.float32) for _ in range(2)],
        [pltpu.VMEM((M, D2), jnp.float32) for _ in range(2)],
        pltpu.SemaphoreType.DMA,
        pltpu.SemaphoreType.DMA,
        pltpu.SemaphoreType.DMA,
    ],
)
def _rope_gather(t0, t1, t2, posf, out,
                 t0_s, t1_s, t2_s, pi_v, i_v, r0_v, r1_v, r2_v,
                 sem_s, sem_g, sem_w):
    nc = 2
    wid = lax.axis_index("s") * nc + lax.axis_index("c")
    wbase = wid * PER_W

    @pl.when(lax.axis_index("s") == 0)
    def _():
        pltpu.sync_copy(t0, t0_s)
        pltpu.sync_copy(t1, t1_s)
        pltpu.sync_copy(t2, t2_s)

    plsc.subcore_barrier()

    i3 = 3 * lax.iota(jnp.int32, 16)
    vmax = (V0 - 1, V1 - 1, V2 - 1)

    def stage(c, b):
        base3 = 3 * (wbase + c * M)
        return pltpu.async_copy(posf.at[pl.ds(base3, 3 * M)],
                                pi_v[b].at[pl.ds(0, 3 * M)], sem_s)

    def deint(b):
        for g in range(G):
            idx = i3 + (48 * g)
            for a in range(3):
                v = plsc.load_gather(pi_v[b], [idx + a])
                v = jnp.minimum(jnp.maximum(v, 0), vmax[a])
                i_v[3 * b + a][pl.ds(16 * g, 16)] = v

    def gathers(b):
        return (pltpu.async_copy(t0_s.at[i_v[3 * b + 0].at[pl.ds(0, M)]],
                                 r0_v[b], sem_g),
                pltpu.async_copy(t1_s.at[i_v[3 * b + 1].at[pl.ds(0, M)]],
                                 r1_v[b], sem_g),
                pltpu.async_copy(t2_s.at[i_v[3 * b + 2].at[pl.ds(0, M)]],
                                 r2_v[b], sem_g))

    def writes(c, b):
        base = wbase + c * M
        return (
            pltpu.async_copy(r0_v[b],
                             out.at[pl.ds(base, M), pl.ds(0, D0)], sem_w),
            pltpu.async_copy(r1_v[b],
                             out.at[pl.ds(base, M), pl.ds(D0, D1)], sem_w),
            pltpu.async_copy(r2_v[b],
                             out.at[pl.ds(base, M), pl.ds(D0 + D1, D2)], sem_w),
        )

    st = [None] * STEPS
    gs = [None] * STEPS
    wr = [None] * STEPS
    st[0] = stage(0, 0)
    st[0].wait()
    deint(0)
    gs[0] = gathers(0)
    if STEPS > 1:
        st[1] = stage(1, 1)
    for c in range(STEPS):
        b = c & 1
        if c + 1 < STEPS:
            st[c + 1].wait()
            deint(1 - b)
        if c >= 1:
            for w in wr[c - 1]:
                w.wait()
        for g2 in gs[c]:
            g2.wait()
        wr[c] = writes(c, b)
        if c + 1 < STEPS:
            gs[c + 1] = gathers(1 - b)
        if c + 2 < STEPS:
            st[c + 2] = stage(c + 2, b)
    for w in wr[STEPS - 1]:
        w.wait()


def kernel(encoder_hidden_states, freqs0, freqs1, freqs2, position_ids):
    del encoder_hidden_states
    t0 = freqs0.reshape(freqs0.shape[0], D0)
    t1 = freqs1.reshape(freqs1.shape[0], D1)
    t2 = freqs2.reshape(freqs2.shape[0], D2)
    posf = position_ids.reshape(3 * N).astype(jnp.int32)
    out = _rope_gather(t0, t1, t2, posf)
    return out.reshape(B, SEQ, DT // 2, 2)
